# Initial kernel scaffold; baseline (speedup 1.0000x reference)
import jax
import jax.numpy as jnp
from jax.experimental import pallas as pl


def kernel(x, edge_index, edge_weight, W1, W2):
    raise NotImplementedError("write your pallas kernel here")


# speedup vs baseline: 4.9987x; 4.9987x over previous
import functools

import jax
import jax.numpy as jnp
from jax import lax
from jax.experimental import pallas as pl
from jax.experimental.pallas import tpu as pltpu
from jax.experimental.pallas import tpu_sc as plsc

NC = 2
NS = 16
LANES = 16

K = 80
SCH = 5
ZCH = 8


def _mm_body(x_ref, w_ref, o_ref):
    o_ref[...] = jnp.dot(x_ref[...], w_ref[...], preferred_element_type=jnp.float32)


def _matmul(x, w, block_rows):
    n, d = x.shape
    f = w.shape[1]
    return pl.pallas_call(
        _mm_body,
        grid=(n // block_rows,),
        in_specs=[
            pl.BlockSpec((block_rows, d), lambda i: (i, 0)),
            pl.BlockSpec((d, f), lambda i: (0, 0)),
        ],
        out_specs=pl.BlockSpec((block_rows, f), lambda i: (i, 0)),
        out_shape=jax.ShapeDtypeStruct((n, f), jnp.float32),
    )(x, w)


def _relu_sum_body(p_ref, o_ref):
    o_ref[...] = jnp.maximum(p_ref[0] + p_ref[1], 0.0)


def _relu_sum(p, block_rows):
    _, n, f = p.shape
    return pl.pallas_call(
        _relu_sum_body,
        grid=(n // block_rows,),
        in_specs=[pl.BlockSpec((2, block_rows, f), lambda i: (0, i, 0))],
        out_specs=pl.BlockSpec((block_rows, f), lambda i: (i, 0)),
        out_shape=jax.ShapeDtypeStruct((n, f), jnp.float32),
    )(p)


def _sum_mm_body(p_ref, w_ref, o_ref):
    o_ref[...] = jnp.dot(p_ref[0] + p_ref[1], w_ref[...],
                         preferred_element_type=jnp.float32)


def _sum_matmul(p, w, nrows, block_rows):
    _, _, d = p.shape
    f = w.shape[1]
    return pl.pallas_call(
        _sum_mm_body,
        grid=(nrows // block_rows,),
        in_specs=[
            pl.BlockSpec((2, block_rows, d), lambda i: (0, i, 0)),
            pl.BlockSpec((d, f), lambda i: (0, 0)),
        ],
        out_specs=pl.BlockSpec((block_rows, f), lambda i: (i, 0)),
        out_shape=jax.ShapeDtypeStruct((nrows, f), jnp.float32),
    )(p, w)


@functools.partial(jax.jit, static_argnames=("npad", "f", "nsc"))
def _spmm(feat, src4, dst4, w4, npad, f, nsc):
    rpt = npad // NS
    nsc2 = nsc // NC

    mesh = plsc.VectorSubcoreMesh(core_axis_name="c", subcore_axis_name="s")

    @functools.partial(
        pl.kernel,
        out_type=jax.ShapeDtypeStruct((NC, npad, f), jnp.float32),
        mesh=mesh,
        scratch_types=[
            pltpu.VMEM((SCH, K), jnp.int32),
            pltpu.VMEM((SCH, K), jnp.int32),
            pltpu.VMEM((SCH, K), jnp.float32),
            pltpu.VMEM((K, f), jnp.float32),
            pltpu.VMEM_SHARED((npad, f), jnp.float32),
            pltpu.SemaphoreType.DMA,
        ],
    )
    def spmm(feat_h, src_h, dst_h, w_h, out_h, src_v, dst_v, w_v, rows_v,
             acc_sh, sem):
        cid = lax.axis_index("c")
        sid = lax.axis_index("s")

        @pl.loop(0, K)
        def _zrow(i):
            for j in range(f // LANES):
                rows_v[i, pl.ds(j * LANES, LANES)] = jnp.zeros((LANES,),
                                                               jnp.float32)

        @pl.loop(0, rpt, step=K)
        def _zcopy(r0):
            pltpu.sync_copy(rows_v, acc_sh.at[pl.ds(sid * rpt + r0, K)])

        plsc.subcore_barrier()

        @pl.loop(0, nsc2)
        def _super(si):
            sc = cid * nsc2 + si
            pltpu.sync_copy(src_h.at[sid, sc], src_v)
            pltpu.sync_copy(dst_h.at[sid, sc], dst_v)
            pltpu.sync_copy(w_h.at[sid, sc], w_v)

            @pl.loop(0, SCH)
            def _chunk(ci):
                pltpu.sync_copy(feat_h.at[src_v.at[ci]], rows_v)

                @pl.loop(0, K // LANES)
                def _grp(gi):
                    wv = w_v[ci, pl.ds(gi * LANES, LANES)]
                    for el in range(LANES):
                        wsplat = jnp.zeros((LANES,), jnp.float32) + wv[el]
                        ei = gi * LANES + el
                        for j in range(f // LANES):
                            sl = pl.ds(j * LANES, LANES)
                            rows_v[ei, sl] = rows_v[ei, sl] * wsplat

                pltpu.sync_copy(rows_v, acc_sh.at[dst_v.at[ci]], add=True)

        plsc.subcore_barrier()

        @pl.loop(0, rpt, step=K)
        def _out(r0):
            pltpu.sync_copy(acc_sh.at[pl.ds(sid * rpt + r0, K)],
                            out_h.at[cid, pl.ds(sid * rpt + r0, K)])

    return spmm(feat, src4, dst4, w4)


def kernel(x, edge_index, edge_weight, W1, W2):
    n, d = x.shape
    h = W1.shape[1]
    e = edge_weight.shape[0]

    nsc = e // (NS * SCH * K)

    npad = -(-n // (NS * K)) * (NS * K)

    src4 = edge_index[0].reshape(NS, nsc, SCH, K)
    dst4 = edge_index[1].reshape(NS, nsc, SCH, K)
    w4 = edge_weight.reshape(NS, nsc, SCH, K)

    pre1 = _matmul(x, W1, 1000)
    p = _spmm(pre1, src4, dst4, w4, npad, h, nsc)
    hh = _relu_sum(p, 1280)
    g = _spmm(hh, src4, dst4, w4, npad, h, nsc)
    return _sum_matmul(g, W2, n, 1000)


# speedup vs baseline: 9.2257x; 1.8456x over previous
import functools

import jax
import jax.numpy as jnp
from jax import lax
from jax.experimental import pallas as pl
from jax.experimental.pallas import tpu as pltpu
from jax.experimental.pallas import tpu_sc as plsc

NC = 2
NS = 16
NTC = NC * NS
LANES = 16

K = 80
NCHK = 128
NGRP = NCHK // 8


def _mm_body(x_ref, w_ref, o_ref):
    o_ref[...] = jnp.dot(x_ref[...], w_ref[...], preferred_element_type=jnp.float32)


def _matmul(x, w, block_rows):
    n, d = x.shape
    f = w.shape[1]
    return pl.pallas_call(
        _mm_body,
        grid=(n // block_rows,),
        in_specs=[
            pl.BlockSpec((block_rows, d), lambda i: (i, 0)),
            pl.BlockSpec((d, f), lambda i: (0, 0)),
        ],
        out_specs=pl.BlockSpec((block_rows, f), lambda i: (i, 0)),
        out_shape=jax.ShapeDtypeStruct((n, f), jnp.float32),
    )(x, w)


def _relu_sum_body(p_ref, o_ref):
    o_ref[...] = jnp.maximum(p_ref[0] + p_ref[1], 0.0)


def _relu_sum(p, block_rows):
    _, n, f = p.shape
    return pl.pallas_call(
        _relu_sum_body,
        grid=(n // block_rows,),
        in_specs=[pl.BlockSpec((2, block_rows, f), lambda i: (0, i, 0))],
        out_specs=pl.BlockSpec((block_rows, f), lambda i: (i, 0)),
        out_shape=jax.ShapeDtypeStruct((n, f), jnp.float32),
    )(p)


def _sum_mm_body(p_ref, w_ref, o_ref):
    o_ref[...] = jnp.dot(p_ref[0] + p_ref[1], w_ref[...],
                         preferred_element_type=jnp.float32)


def _sum_matmul(p, w, nrows, block_rows):
    _, _, d = p.shape
    f = w.shape[1]
    return pl.pallas_call(
        _sum_mm_body,
        grid=(nrows // block_rows,),
        in_specs=[
            pl.BlockSpec((2, block_rows, d), lambda i: (0, i, 0)),
            pl.BlockSpec((d, f), lambda i: (0, 0)),
        ],
        out_specs=pl.BlockSpec((block_rows, f), lambda i: (i, 0)),
        out_shape=jax.ShapeDtypeStruct((nrows, f), jnp.float32),
    )(p, w)


@functools.partial(jax.jit, static_argnames=("npad", "f"))
def _spmm(feat, srcf, dst4, wf, npad, f):
    rpt = npad // NS
    epw = NCHK * K

    mesh = plsc.VectorSubcoreMesh(core_axis_name="c", subcore_axis_name="s")

    @functools.partial(
        pl.kernel,
        out_type=jax.ShapeDtypeStruct((NC, npad, f), jnp.float32),
        mesh=mesh,
        scratch_types=[
            pltpu.VMEM((epw,), jnp.int32),
            pltpu.VMEM((epw,), jnp.float32),
            pltpu.VMEM((8, K), jnp.int32),
            pltpu.VMEM((8, K), jnp.int32),
            pltpu.VMEM((K, f), jnp.float32),
            pltpu.VMEM((K, f), jnp.float32),
            pltpu.VMEM_SHARED((npad, f), jnp.float32),
            pltpu.SemaphoreType.DMA,
            pltpu.SemaphoreType.DMA,
            pltpu.SemaphoreType.DMA,
            pltpu.SemaphoreType.DMA,
            pltpu.SemaphoreType.DMA,
            pltpu.SemaphoreType.DMA,
        ],
    )
    def spmm(feat_h, src_h, dst_h, w_h, out_h, src_v, w_v, dstb0, dstb1,
             rows0, rows1, acc_sh, sg0, sg1, ss0, ss1, st0, st1):
        cid = lax.axis_index("c")
        sid = lax.axis_index("s")
        tc = cid * NS + sid
        rows = (rows0, rows1)
        dstb = (dstb0, dstb1)
        sg = (sg0, sg1)
        ss = (ss0, ss1)
        st = (st0, st1)

        pltpu.sync_copy(src_h.at[pl.ds(tc * epw, epw)], src_v)
        pltpu.sync_copy(w_h.at[pl.ds(tc * epw, epw)], w_v)

        @pl.loop(0, K)
        def _zrow(i):
            for j in range(f // LANES):
                z = jnp.zeros((LANES,), jnp.float32)
                rows0[i, pl.ds(j * LANES, LANES)] = z
                rows1[i, pl.ds(j * LANES, LANES)] = z

        @pl.loop(0, rpt, step=K)
        def _zcopy(r0):
            pltpu.sync_copy(rows0, acc_sh.at[pl.ds(sid * rpt + r0, K)])

        plsc.subcore_barrier()

        def gather(ci, b):
            return pltpu.make_async_copy(
                feat_h.at[src_v.at[pl.ds(ci * K, K)]], rows[b], sg[b])

        def scatter(k, b, gb):
            return pltpu.make_async_copy(
                rows[b], acc_sh.at[dstb[gb].at[k]], ss[b])

        def scatter_start(k, b, gb):
            pltpu.async_copy(rows[b], acc_sh.at[dstb[gb].at[k]], ss[b],
                             add=True)

        def stage(grp, gb):
            return pltpu.make_async_copy(dst_h.at[tc, grp], dstb[gb], st[gb])

        stage(0, 0).start()
        gather(0, 0).start()

        @pl.loop(0, NCHK)
        def _chunk(ci):
            for b in (0, 1):
                for gb in (0, 1):

                    @pl.when((ci & 9) == (b | (gb << 3)))
                    def _body():
                        nb = 1 - b

                        @pl.when(ci >= 1)
                        def _ws():
                            scatter(0, nb, gb).wait()

                        gather((ci + 1) & (NCHK - 1), nb).start()

                        @pl.when((ci & 7) == 0)
                        def _grpstart():
                            stage(0, gb).wait()
                            stage(((ci >> 3) + 1) & (NGRP - 1), 1 - gb).start()

                        gather(ci, b).wait()

                        @pl.loop(0, K // LANES)
                        def _grp(gi):
                            wv = w_v[pl.ds(ci * K + gi * LANES, LANES)]
                            for el in range(LANES):
                                wsplat = jnp.zeros((LANES,), jnp.float32) + wv[el]
                                ei = gi * LANES + el
                                for j in range(f // LANES):
                                    sl = pl.ds(j * LANES, LANES)
                                    rows[b][ei, sl] = rows[b][ei, sl] * wsplat

                        scatter_start(ci & 7, b, gb)

        scatter(0, 1, 0).wait()
        gather(0, 0).wait()
        stage(0, 0).wait()

        plsc.subcore_barrier()

        @pl.loop(0, rpt, step=K)
        def _out(r0):
            pltpu.sync_copy(acc_sh.at[pl.ds(sid * rpt + r0, K)],
                            out_h.at[cid, pl.ds(sid * rpt + r0, K)])

    return spmm(feat, srcf, dst4, wf)


def kernel(x, edge_index, edge_weight, W1, W2):
    n, d = x.shape
    h = W1.shape[1]
    e = edge_weight.shape[0]

    npad = -(-n // (NS * K)) * (NS * K)

    ep = NTC * NCHK * K
    pad = ep - e
    idx = jnp.arange(pad, dtype=jnp.int32)
    srcf = jnp.concatenate([edge_index[0], idx % n])
    dstf = jnp.concatenate([edge_index[1], idx % npad])
    wf = jnp.concatenate([edge_weight, jnp.zeros((pad,), jnp.float32)])
    dst4 = dstf.reshape(NTC, NGRP, 8, K)

    pre1 = _matmul(x, W1, 1000)
    p = _spmm(pre1, srcf, dst4, wf, npad, h)
    hh = _relu_sum(p, 1280)
    g = _spmm(hh, srcf, dst4, wf, npad, h)
    return _sum_matmul(g, W2, n, 1000)


# speedup vs baseline: 10.5559x; 1.1442x over previous
import functools

import jax
import jax.numpy as jnp
from jax import lax
from jax.experimental import pallas as pl
from jax.experimental.pallas import tpu as pltpu
from jax.experimental.pallas import tpu_sc as plsc

NC = 2
NS = 16
NTC = NC * NS
LANES = 16

K = 80
NCHK = 128
NGRP = NCHK // 8


def _mm_body(x_ref, w_ref, o_ref):
    o_ref[...] = jnp.dot(x_ref[...], w_ref[...], preferred_element_type=jnp.float32)


def _matmul(x, w, block_rows):
    n, d = x.shape
    f = w.shape[1]
    return pl.pallas_call(
        _mm_body,
        grid=(n // block_rows,),
        in_specs=[
            pl.BlockSpec((block_rows, d), lambda i: (i, 0)),
            pl.BlockSpec((d, f), lambda i: (0, 0)),
        ],
        out_specs=pl.BlockSpec((block_rows, f), lambda i: (i, 0)),
        out_shape=jax.ShapeDtypeStruct((n, f), jnp.float32),
    )(x, w)


def _relu_sum_body(p_ref, o_ref):
    o_ref[...] = jnp.maximum(p_ref[0] + p_ref[1], 0.0)


def _relu_sum(p, block_rows):
    _, n, f = p.shape
    return pl.pallas_call(
        _relu_sum_body,
        grid=(n // block_rows,),
        in_specs=[pl.BlockSpec((2, block_rows, f), lambda i: (0, i, 0))],
        out_specs=pl.BlockSpec((block_rows, f), lambda i: (i, 0)),
        out_shape=jax.ShapeDtypeStruct((n, f), jnp.float32),
    )(p)


def _sum_mm_body(p_ref, w_ref, o_ref):
    o_ref[...] = jnp.dot(p_ref[0] + p_ref[1], w_ref[...],
                         preferred_element_type=jnp.float32)


def _sum_matmul(p, w, nrows, block_rows):
    _, _, d = p.shape
    f = w.shape[1]
    return pl.pallas_call(
        _sum_mm_body,
        grid=(nrows // block_rows,),
        in_specs=[
            pl.BlockSpec((2, block_rows, d), lambda i: (0, i, 0)),
            pl.BlockSpec((d, f), lambda i: (0, 0)),
        ],
        out_specs=pl.BlockSpec((block_rows, f), lambda i: (i, 0)),
        out_shape=jax.ShapeDtypeStruct((nrows, f), jnp.float32),
    )(p, w)


NBUF = 4


@functools.partial(jax.jit, static_argnames=("npad", "f"))
def _spmm(feat, src4, dst4, w4, npad, f):
    rpt = npad // NS

    mesh = plsc.VectorSubcoreMesh(core_axis_name="c", subcore_axis_name="s")

    @functools.partial(
        pl.kernel,
        out_type=jax.ShapeDtypeStruct((NC, npad, f), jnp.float32),
        mesh=mesh,
        scratch_types=[
            pltpu.VMEM((8, K), jnp.int32),
            pltpu.VMEM((8, K), jnp.int32),
            pltpu.VMEM((8, K), jnp.int32),
            pltpu.VMEM((8, K), jnp.int32),
            pltpu.VMEM((8, K), jnp.float32),
            pltpu.VMEM((8, K), jnp.float32),
            pltpu.VMEM((K, f), jnp.float32),
            pltpu.VMEM((K, f), jnp.float32),
            pltpu.VMEM((K, f), jnp.float32),
            pltpu.VMEM((K, f), jnp.float32),
            pltpu.VMEM_SHARED((npad, f), jnp.float32),
            pltpu.SemaphoreType.DMA,
            pltpu.SemaphoreType.DMA,
            pltpu.SemaphoreType.DMA,
            pltpu.SemaphoreType.DMA,
            pltpu.SemaphoreType.DMA,
            pltpu.SemaphoreType.DMA,
            pltpu.SemaphoreType.DMA,
            pltpu.SemaphoreType.DMA,
            pltpu.SemaphoreType.DMA,
            pltpu.SemaphoreType.DMA,
            pltpu.SemaphoreType.DMA,
        ],
    )
    def spmm(feat_h, src_h, dst_h, w_h, out_h, srcb0, srcb1, dstb0, dstb1,
             wb0, wb1, rows0, rows1, rows2, rows3, acc_sh,
             sg0, sg1, sg2, sg3, ss0, ss1, ss2, ss3, st0, st1, sz):
        cid = lax.axis_index("c")
        sid = lax.axis_index("s")
        tc = cid * NS + sid
        rows = (rows0, rows1, rows2, rows3)
        srcb = (srcb0, srcb1)
        dstb = (dstb0, dstb1)
        wb = (wb0, wb1)
        sg = (sg0, sg1, sg2, sg3)
        ss = (ss0, ss1, ss2, ss3)
        st = (st0, st1)

        @pl.loop(0, K)
        def _zrow(i):
            for j in range(f // LANES):
                rows0[i, pl.ds(j * LANES, LANES)] = jnp.zeros((LANES,),
                                                              jnp.float32)

        @pl.loop(0, rpt, step=K)
        def _zcopy(r0):
            pltpu.async_copy(rows0, acc_sh.at[pl.ds(sid * rpt + r0, K)], sz)

        @pl.loop(0, rpt, step=K)
        def _zwait(r0):
            pltpu.make_async_copy(
                rows0, acc_sh.at[pl.ds(sid * rpt + r0, K)], sz).wait()

        plsc.subcore_barrier()

        def gather(idxref, b):
            return pltpu.make_async_copy(feat_h.at[idxref], rows[b], sg[b])

        def stage_descs(grp, p):
            return (
                pltpu.make_async_copy(src_h.at[tc, grp], srcb[p], st[p]),
                pltpu.make_async_copy(dst_h.at[tc, grp], dstb[p], st[p]),
                pltpu.make_async_copy(w_h.at[tc, grp], wb[p], st[p]),
            )

        pltpu.sync_copy(src_h.at[tc, 0], srcb0)
        pltpu.sync_copy(dst_h.at[tc, 0], dstb0)
        pltpu.sync_copy(w_h.at[tc, 0], wb0)
        for j in range(NBUF - 1):
            gather(srcb[0].at[j], j).start()

        def chunk_iter(g, k, gb):
            b = k % NBUF
            bp = (k - 1) % NBUF

            if k == 5:
                for dsc in stage_descs(0, 1 - gb):
                    dsc.wait()

            gather(srcb[gb].at[k], b).wait()

            @pl.loop(0, K // LANES)
            def _grp(gi):
                wv = wb[gb][k, pl.ds(gi * LANES, LANES)]
                for el in range(LANES):
                    wsplat = jnp.zeros((LANES,), jnp.float32) + wv[el]
                    ei = gi * LANES + el
                    for j in range(f // LANES):
                        sl = pl.ds(j * LANES, LANES)
                        rows[b][ei, sl] = rows[b][ei, sl] * wsplat

            pltpu.async_copy(rows[b], acc_sh.at[dstb[gb].at[k]], ss[b],
                             add=True)

            def _wait_prev():
                pltpu.make_async_copy(rows[bp], acc_sh.at[dstb[gb].at[0]],
                                      ss[bp]).wait()
            if k == 0:
                pl.when(g >= 1)(_wait_prev)
            else:
                _wait_prev()

            if k <= 4:
                idxref = srcb[gb].at[k + 3]
            else:
                idxref = srcb[1 - gb].at[k - 5]
            gather(idxref, bp).start()

            if k == 0:
                for dsc in stage_descs((g + 1) & (NGRP - 1), 1 - gb):
                    dsc.start()

        @pl.loop(0, NGRP)
        def _group(g):
            for gb in (0, 1):

                @pl.when((g & 1) == gb)
                def _body():
                    for k in range(8):
                        chunk_iter(g, k, gb)

        pltpu.make_async_copy(rows[3], acc_sh.at[dstb[0].at[0]], ss[3]).wait()
        for j in range(NBUF - 1):
            gather(srcb[0].at[0], j).wait()

        plsc.subcore_barrier()

        @pl.loop(0, rpt, step=K)
        def _out(r0):
            pltpu.async_copy(acc_sh.at[pl.ds(sid * rpt + r0, K)],
                             out_h.at[cid, pl.ds(sid * rpt + r0, K)], sz)

        @pl.loop(0, rpt, step=K)
        def _owait(r0):
            pltpu.make_async_copy(
                acc_sh.at[pl.ds(sid * rpt + r0, K)],
                out_h.at[cid, pl.ds(sid * rpt + r0, K)], sz).wait()

    return spmm(feat, src4, dst4, w4)


def kernel(x, edge_index, edge_weight, W1, W2):
    n, d = x.shape
    h = W1.shape[1]
    e = edge_weight.shape[0]

    npad = -(-n // (NS * K)) * (NS * K)

    ep = NTC * NCHK * K
    pad = ep - e
    idx = jnp.arange(pad, dtype=jnp.int32)
    src4 = jnp.concatenate([edge_index[0], idx % n]).reshape(NTC, NGRP, 8, K)
    dst4 = jnp.concatenate([edge_index[1], idx % npad]).reshape(NTC, NGRP, 8, K)
    w4 = jnp.concatenate([edge_weight, jnp.zeros((pad,), jnp.float32)]
                         ).reshape(NTC, NGRP, 8, K)

    pre1 = _matmul(x, W1, 1000)
    p = _spmm(pre1, src4, dst4, w4, npad, h)
    hh = _relu_sum(p, 1280)
    g = _spmm(hh, src4, dst4, w4, npad, h)
    return _sum_matmul(g, W2, n, 1000)
